# baseline (speedup 1.0000x reference)
import functools

import jax
import jax.numpy as jnp
from jax import lax
from jax.experimental import layout as jax_layout
from jax.experimental import pallas as pl
from jax.experimental.pallas import tpu as pltpu
from jax.experimental.pallas import tpu_sc as plsc

_NW = 32
_SUB = 128
_NSUB = 16
_CH = _SUB * _NSUB


def _gather_body(table_hbm, idx_hbm, out_hbm, idx_v, rows_v, sem, *, B, C):
    c = lax.axis_index("c")
    s = lax.axis_index("s")
    wid = s * 2 + c
    nchunks = (B + _CH - 1) // _CH

    def chunk(t, carry):
        j = wid + t * _NW
        start = jnp.minimum(j * _CH, B - _CH)
        pltpu.sync_copy(idx_hbm.at[pl.ds(start, _CH)], idx_v)
        copies = []
        for i in range(_NSUB):
            copies.append(pltpu.async_copy(
                table_hbm.at[idx_v.at[pl.ds(i * _SUB, _SUB)]],
                rows_v.at[pl.ds(i * _SUB, _SUB)], sem))
        for cp in copies:
            cp.wait()
        pltpu.sync_copy(rows_v, out_hbm.at[pl.ds(start, _CH)])
        return carry

    n_mine = jnp.maximum(0, (nchunks - wid + _NW - 1) // _NW)
    lax.fori_loop(0, n_mine, chunk, 0)


def _sc_gather(table, idx):
    B = idx.shape[0]
    C = table.shape[1]
    mesh = plsc.VectorSubcoreMesh(core_axis_name="c", subcore_axis_name="s")
    k = pl.kernel(
        functools.partial(_gather_body, B=B, C=C),
        out_type=jax.ShapeDtypeStruct((B, C), jnp.float32),
        mesh=mesh,
        scratch_types=[
            pltpu.VMEM((_CH,), jnp.int32),
            pltpu.VMEM((_CH, C), jnp.float32),
            pltpu.SemaphoreType.DMA,
        ],
        compiler_params=pltpu.CompilerParams(use_tc_tiling_on_sc=False),
    )
    return k(table, idx)


def _stage(feat, nbr, W, b, Wc, bc, num):
    n, k = nbr.shape
    cin = feat.shape[1]
    g = _sc_gather(feat, nbr.reshape(-1)).reshape(n, k, cin)
    h = jax.nn.relu(jnp.einsum('nkc,kcd->nd', g, W) + b)
    h = jax_layout.with_layout_constraint(
        h, jax_layout.Layout(major_to_minor=(1, 0)))
    cout = h.shape[1]
    gc = _sc_gather(h, nbr.reshape(-1)).reshape(n, k, cout)
    cls = jnp.einsum('nkc,kcd->nd', gc, Wc) + bc
    idx = lax.top_k(cls[:, 0], num)[1]
    return h, cls, idx


def _upsample(h_sel, Wu, bu):
    out = jnp.einsum('nc,kcd->nkd', h_sel, Wu) + bu
    return jax.nn.relu(out.reshape(-1, out.shape[-1]))


def kernel(x, nbr0, nbr1, nbr2, nums0, nums1, nums2,
           W0, b0, W0c, b0c, Wu1, bu1, W1, b1, W1c, b1c,
           Wu2, bu2, W2, b2, W2c, b2c):
    del nums0, nums1, nums2
    NUM0, NUM1, NUM2 = 16384, 16384, 65536
    h, cls0, idx0 = _stage(x, nbr0, W0, b0, W0c, b0c, NUM0)
    h = _sc_gather(h, idx0)
    h, cls1, idx1 = _stage(_upsample(h, Wu1, bu1), nbr1, W1, b1, W1c, b1c, NUM1)
    h = _sc_gather(h, idx1)
    h, cls2, idx2 = _stage(_upsample(h, Wu2, bu2), nbr2, W2, b2, W2c, b2c, NUM2)
    out = _sc_gather(h, idx2)
    return (cls0, cls1, cls2, out)
